# baseline (speedup 1.0000x reference)
import functools

import jax
import jax.numpy as jnp
from jax import lax
from jax.experimental import pallas as pl
from jax.experimental.pallas import tpu as pltpu
from jax.experimental.pallas import tpu_sc as plsc

B = 16384
D = 512
DQ = 256
K = 8192

TB = 256
WINDOWS = [(0, 2736), (2736, 5472), (5472, 8192)]


def _zq_body(ze_ref, winT_ref, bin_ref, zq_ref):
    zq_ref[...] = jnp.dot(ze_ref[...], winT_ref[...],
                          preferred_element_type=jnp.float32) + bin_ref[...]


def _zq_call(ze, winT, bin2d):
    return pl.pallas_call(
        _zq_body,
        grid=(B // TB,),
        in_specs=[
            pl.BlockSpec((TB, D), lambda i: (i, 0)),
            pl.BlockSpec((D, DQ), lambda i: (0, 0)),
            pl.BlockSpec((1, DQ), lambda i: (0, 0)),
        ],
        out_specs=pl.BlockSpec((TB, DQ), lambda i: (i, 0)),
        out_shape=jax.ShapeDtypeStruct((B, DQ), jnp.float32),
    )(ze, winT, bin2d)


def _argmin_body(zqT_ref, cb_ref, out_ref):
    zqT = zqT_ref[...]
    zsq = jnp.sum(zqT * zqT, axis=0, keepdims=True)
    cbm = cb_ref[...]
    csq = jnp.sum(cbm * cbm, axis=1, keepdims=True)
    scores = lax.dot_general(cbm, zqT, (((1,), (0,)), ((), ())),
                             preferred_element_type=jnp.float32)
    dist = (zsq + csq) - 2.0 * scores
    kio = lax.broadcasted_iota(jnp.int32, (K, TB), 0)
    acc_v = jnp.full((1, TB), jnp.inf, jnp.float32)
    acc_i = jnp.zeros((1, TB), jnp.int32)
    for lo, hi in WINDOWS:
        m = (kio >= lo) & (kio < hi)
        dm = jnp.where(m, dist, jnp.inf)
        mw = jnp.min(dm, axis=0, keepdims=True)
        iw = jnp.min(jnp.where(dm == mw, kio, K), axis=0, keepdims=True)
        upd = mw < acc_v
        acc_i = jnp.where(upd, iw, acc_i)
        mwq = mw.astype(jnp.bfloat16).astype(jnp.float32)
        acc_v = jnp.where(upd, mwq, acc_v)
    out_ref[...] = acc_i.reshape(1, 1, TB)


def _argmin_call(zqT, codebook):
    out = pl.pallas_call(
        _argmin_body,
        grid=(B // TB,),
        in_specs=[
            pl.BlockSpec((DQ, TB), lambda i: (0, i)),
            pl.BlockSpec((K, DQ), lambda i: (0, 0)),
        ],
        out_specs=pl.BlockSpec((1, 1, TB), lambda i: (i, 0, 0)),
        out_shape=jax.ShapeDtypeStruct((B // TB, 1, TB), jnp.int32),
    )(zqT, codebook)
    return out.reshape(B)


CB = 512


def _cbproj_body(cb_ref, woutT_ref, bout_ref, p_ref):
    p_ref[...] = jnp.dot(cb_ref[...], woutT_ref[...],
                         preferred_element_type=jnp.float32) + bout_ref[...]


def _cbproj_call(codebook, woutT, bout2d):
    return pl.pallas_call(
        _cbproj_body,
        grid=(K // CB,),
        in_specs=[
            pl.BlockSpec((CB, DQ), lambda i: (i, 0)),
            pl.BlockSpec((DQ, D), lambda i: (0, 0)),
            pl.BlockSpec((1, D), lambda i: (0, 0)),
        ],
        out_specs=pl.BlockSpec((CB, D), lambda i: (i, 0)),
        out_shape=jax.ShapeDtypeStruct((K, D), jnp.float32),
    )(codebook, woutT, bout2d)


NC = 2
NS = 16
NW = NC * NS
BPW = B // NW
CH = 128


def _gather_body(idx_hbm, cb_hbm, p_hbm, qq_hbm, qz_hbm,
                 idx_v, rows1, rows2, sem1, sem2):
    wid = lax.axis_index("s") * NC + lax.axis_index("c")
    base = wid * BPW
    for c in range(BPW // CH):
        off = base + c * CH
        pltpu.sync_copy(idx_hbm.at[pl.ds(off, CH)], idx_v)
        cp1 = pltpu.async_copy(cb_hbm.at[idx_v], rows1, sem1)
        cp2 = pltpu.async_copy(p_hbm.at[idx_v], rows2, sem2)
        cp1.wait()
        pltpu.sync_copy(rows1, qq_hbm.at[pl.ds(off, CH)])
        cp2.wait()
        pltpu.sync_copy(rows2, qz_hbm.at[pl.ds(off, CH)])


@functools.cache
def _gather_call():
    return pl.kernel(
        _gather_body,
        mesh=plsc.VectorSubcoreMesh(core_axis_name="c", subcore_axis_name="s"),
        out_type=[
            jax.ShapeDtypeStruct((B, DQ), jnp.float32),
            jax.ShapeDtypeStruct((B, D), jnp.float32),
        ],
        scratch_types=[
            pltpu.VMEM((CH,), jnp.int32),
            pltpu.VMEM((CH, DQ), jnp.float32),
            pltpu.VMEM((CH, D), jnp.float32),
            pltpu.SemaphoreType.DMA,
            pltpu.SemaphoreType.DMA,
        ],
    )


def kernel(ze, W_in, b_in, codebook, W_out, b_out):
    zq = _zq_call(ze, W_in.T, b_in.reshape(1, DQ))
    idx = _argmin_call(zq.T, codebook)
    proj_cb = _cbproj_call(codebook, W_out.T, b_out.reshape(1, D))
    qq, qz = _gather_call()(idx, codebook, proj_cb)
    zero = jnp.float32(0.0)
    return (idx[:, None], qz, qq, qq[:, None, :], zero, zero, zero)


# speedup vs baseline: 1.4531x; 1.4531x over previous
import functools

import jax
import jax.numpy as jnp
from jax import lax
from jax.experimental import pallas as pl
from jax.experimental.pallas import tpu as pltpu
from jax.experimental.pallas import tpu_sc as plsc

B = 16384
D = 512
DQ = 256
K = 8192

TB = 256
WINDOWS = [(0, 2736), (2736, 5472), (5472, 8192)]


def _zq_body(ze_ref, winT_ref, bin_ref, zqT_ref):
    zq = jnp.dot(ze_ref[...], winT_ref[...],
                 preferred_element_type=jnp.float32) + bin_ref[...]
    zqT_ref[...] = zq.T


def _zq_call(ze, winT, bin2d):
    return pl.pallas_call(
        _zq_body,
        grid=(B // TB,),
        in_specs=[
            pl.BlockSpec((TB, D), lambda i: (i, 0)),
            pl.BlockSpec((D, DQ), lambda i: (0, 0)),
            pl.BlockSpec((1, DQ), lambda i: (0, 0)),
        ],
        out_specs=pl.BlockSpec((DQ, TB), lambda i: (0, i)),
        out_shape=jax.ShapeDtypeStruct((DQ, B), jnp.float32),
    )(ze, winT, bin2d)


def _argmin_body(zqT_ref, cb_ref, out_ref):
    zqT = zqT_ref[...]
    zsq = jnp.sum(zqT * zqT, axis=0, keepdims=True)
    zq2 = zqT + zqT
    acc_v = jnp.full((1, TB), jnp.inf, jnp.float32)
    acc_i = jnp.zeros((1, TB), jnp.int32)
    for lo, hi in WINDOWS:
        cbw = cb_ref[pl.ds(lo, hi - lo), :]
        csq = jnp.sum(cbw * cbw, axis=1, keepdims=True)
        s2 = lax.dot_general(cbw, zq2, (((1,), (0,)), ((), ())),
                             preferred_element_type=jnp.float32)
        dist = (zsq + csq) - s2
        kio = lax.broadcasted_iota(jnp.int32, (hi - lo, TB), 0)
        mw = jnp.min(dist, axis=0, keepdims=True)
        iw = jnp.min(jnp.where(dist == mw, kio, K), axis=0, keepdims=True) + lo
        upd = mw < acc_v
        acc_i = jnp.where(upd, iw, acc_i)
        mwq = mw.astype(jnp.bfloat16).astype(jnp.float32)
        acc_v = jnp.where(upd, mwq, acc_v)
    out_ref[...] = acc_i.reshape(1, 1, TB)


def _argmin_call(zqT, codebook):
    out = pl.pallas_call(
        _argmin_body,
        grid=(B // TB,),
        in_specs=[
            pl.BlockSpec((DQ, TB), lambda i: (0, i)),
            pl.BlockSpec((K, DQ), lambda i: (0, 0)),
        ],
        out_specs=pl.BlockSpec((1, 1, TB), lambda i: (i, 0, 0)),
        out_shape=jax.ShapeDtypeStruct((B // TB, 1, TB), jnp.int32),
    )(zqT, codebook)
    return out.reshape(B)


CB = 512


def _cbproj_body(cb_ref, woutT_ref, bout_ref, p_ref):
    p_ref[...] = jnp.dot(cb_ref[...], woutT_ref[...],
                         preferred_element_type=jnp.float32) + bout_ref[...]


def _cbproj_call(codebook, woutT, bout2d):
    return pl.pallas_call(
        _cbproj_body,
        grid=(K // CB,),
        in_specs=[
            pl.BlockSpec((CB, DQ), lambda i: (i, 0)),
            pl.BlockSpec((DQ, D), lambda i: (0, 0)),
            pl.BlockSpec((1, D), lambda i: (0, 0)),
        ],
        out_specs=pl.BlockSpec((CB, D), lambda i: (i, 0)),
        out_shape=jax.ShapeDtypeStruct((K, D), jnp.float32),
    )(codebook, woutT, bout2d)


NC = 2
NS = 16
NW = NC * NS
BPW = B // NW
CH = 128


def _gather_body(idx_hbm, cb_hbm, p_hbm, qq_hbm, qz_hbm,
                 idx_v, rows1, rows2, sem1, sem2):
    wid = lax.axis_index("s") * NC + lax.axis_index("c")
    base = wid * BPW
    for c in range(BPW // CH):
        off = base + c * CH
        pltpu.sync_copy(idx_hbm.at[pl.ds(off, CH)], idx_v)
        cp1 = pltpu.async_copy(cb_hbm.at[idx_v], rows1, sem1)
        cp2 = pltpu.async_copy(p_hbm.at[idx_v], rows2, sem2)
        cp1.wait()
        pltpu.sync_copy(rows1, qq_hbm.at[pl.ds(off, CH)])
        cp2.wait()
        pltpu.sync_copy(rows2, qz_hbm.at[pl.ds(off, CH)])


@functools.cache
def _gather_call():
    return pl.kernel(
        _gather_body,
        mesh=plsc.VectorSubcoreMesh(core_axis_name="c", subcore_axis_name="s"),
        out_type=[
            jax.ShapeDtypeStruct((B, DQ), jnp.float32),
            jax.ShapeDtypeStruct((B, D), jnp.float32),
        ],
        scratch_types=[
            pltpu.VMEM((CH,), jnp.int32),
            pltpu.VMEM((CH, DQ), jnp.float32),
            pltpu.VMEM((CH, D), jnp.float32),
            pltpu.SemaphoreType.DMA,
            pltpu.SemaphoreType.DMA,
        ],
    )


def kernel(ze, W_in, b_in, codebook, W_out, b_out):
    zqT = _zq_call(ze, W_in.T, b_in.reshape(1, DQ))
    idx = _argmin_call(zqT, codebook)
    proj_cb = _cbproj_call(codebook, W_out.T, b_out.reshape(1, D))
    qq, qz = _gather_call()(idx, codebook, proj_cb)
    zero = jnp.float32(0.0)
    return (idx[:, None], qz, qq, qq[:, None, :], zero, zero, zero)


# speedup vs baseline: 1.4608x; 1.0053x over previous
import functools

import jax
import jax.numpy as jnp
from jax import lax
from jax.experimental import pallas as pl
from jax.experimental.pallas import tpu as pltpu
from jax.experimental.pallas import tpu_sc as plsc

B = 16384
D = 512
DQ = 256
K = 8192

TB = 256
WINDOWS = [(0, 2736), (2736, 5472), (5472, 8192)]


def _zq_body(ze_ref, winT_ref, bin_ref, zqT_ref):
    zq = jnp.dot(ze_ref[...], winT_ref[...],
                 preferred_element_type=jnp.float32) + bin_ref[...]
    zqT_ref[...] = zq.T


def _zq_call(ze, winT, bin2d):
    return pl.pallas_call(
        _zq_body,
        grid=(B // TB,),
        in_specs=[
            pl.BlockSpec((TB, D), lambda i: (i, 0)),
            pl.BlockSpec((D, DQ), lambda i: (0, 0)),
            pl.BlockSpec((1, DQ), lambda i: (0, 0)),
        ],
        out_specs=pl.BlockSpec((DQ, TB), lambda i: (0, i)),
        out_shape=jax.ShapeDtypeStruct((DQ, B), jnp.float32),
    )(ze, winT, bin2d)


def _argmin_body(zqT_ref, cb_ref, out_ref):
    zqT = zqT_ref[...]
    zsq = jnp.sum(zqT * zqT, axis=0, keepdims=True)
    zq2 = zqT + zqT
    acc_v = jnp.full((1, TB), jnp.inf, jnp.float32)
    acc_i = jnp.zeros((1, TB), jnp.int32)
    kio_full = lax.broadcasted_iota(jnp.int32, (WINDOWS[0][1], TB), 0)
    for lo, hi in WINDOWS:
        cbw = cb_ref[pl.ds(lo, hi - lo), :]
        csq = jnp.sum(cbw * cbw, axis=1, keepdims=True)
        s2 = lax.dot_general(cbw, zq2, (((1,), (0,)), ((), ())),
                             preferred_element_type=jnp.float32)
        dist = (zsq + csq) - s2
        kio = lax.slice(kio_full, (0, 0), (hi - lo, TB))
        mw = jnp.min(dist, axis=0, keepdims=True)
        iw = jnp.min(jnp.where(dist == mw, kio, K), axis=0, keepdims=True) + lo
        upd = mw < acc_v
        acc_i = jnp.where(upd, iw, acc_i)
        mwq = mw.astype(jnp.bfloat16).astype(jnp.float32)
        acc_v = jnp.where(upd, mwq, acc_v)
    out_ref[...] = acc_i.reshape(1, 1, TB)


def _argmin_call(zqT, codebook):
    out = pl.pallas_call(
        _argmin_body,
        grid=(B // TB,),
        in_specs=[
            pl.BlockSpec((DQ, TB), lambda i: (0, i)),
            pl.BlockSpec((K, DQ), lambda i: (0, 0)),
        ],
        out_specs=pl.BlockSpec((1, 1, TB), lambda i: (i, 0, 0)),
        out_shape=jax.ShapeDtypeStruct((B // TB, 1, TB), jnp.int32),
    )(zqT, codebook)
    return out.reshape(B)


CB = 512


def _cbproj_body(cb_ref, woutT_ref, bout_ref, p_ref):
    p_ref[...] = jnp.dot(cb_ref[...], woutT_ref[...],
                         preferred_element_type=jnp.float32) + bout_ref[...]


def _cbproj_call(codebook, woutT, bout2d):
    return pl.pallas_call(
        _cbproj_body,
        grid=(K // CB,),
        in_specs=[
            pl.BlockSpec((CB, DQ), lambda i: (i, 0)),
            pl.BlockSpec((DQ, D), lambda i: (0, 0)),
            pl.BlockSpec((1, D), lambda i: (0, 0)),
        ],
        out_specs=pl.BlockSpec((CB, D), lambda i: (i, 0)),
        out_shape=jax.ShapeDtypeStruct((K, D), jnp.float32),
    )(codebook, woutT, bout2d)


NC = 2
NS = 16
NW = NC * NS
BPW = B // NW
CH = 64
NCH = BPW // CH


def _gather_body(idx_hbm, cb_hbm, p_hbm, qq_hbm, qz_hbm,
                 idx_v, r1a, r1b, r2a, r2b, s1a, s1b, s2a, s2b):
    wid = lax.axis_index("s") * NC + lax.axis_index("c")
    base = wid * BPW
    rows1, rows2 = [r1a, r1b], [r2a, r2b]
    sem1, sem2 = [s1a, s1b], [s2a, s2b]
    pltpu.sync_copy(idx_hbm.at[pl.ds(base, BPW)], idx_v)
    cps = [None, None]
    for c in range(NCH + 1):
        if c < NCH:
            b = c % 2
            sl = idx_v.at[pl.ds(c * CH, CH)]
            cps[b] = (pltpu.async_copy(cb_hbm.at[sl], rows1[b], sem1[b]),
                      pltpu.async_copy(p_hbm.at[sl], rows2[b], sem2[b]))
        if c >= 1:
            p = (c - 1) % 2
            off = base + (c - 1) * CH
            cps[p][0].wait()
            pltpu.sync_copy(rows1[p], qq_hbm.at[pl.ds(off, CH)])
            cps[p][1].wait()
            pltpu.sync_copy(rows2[p], qz_hbm.at[pl.ds(off, CH)])


@functools.cache
def _gather_call():
    return pl.kernel(
        _gather_body,
        mesh=plsc.VectorSubcoreMesh(core_axis_name="c", subcore_axis_name="s"),
        out_type=[
            jax.ShapeDtypeStruct((B, DQ), jnp.float32),
            jax.ShapeDtypeStruct((B, D), jnp.float32),
        ],
        scratch_types=[
            pltpu.VMEM((BPW,), jnp.int32),
            pltpu.VMEM((CH, DQ), jnp.float32),
            pltpu.VMEM((CH, DQ), jnp.float32),
            pltpu.VMEM((CH, D), jnp.float32),
            pltpu.VMEM((CH, D), jnp.float32),
            pltpu.SemaphoreType.DMA,
            pltpu.SemaphoreType.DMA,
            pltpu.SemaphoreType.DMA,
            pltpu.SemaphoreType.DMA,
        ],
    )


def kernel(ze, W_in, b_in, codebook, W_out, b_out):
    zqT = _zq_call(ze, W_in.T, b_in.reshape(1, DQ))
    idx = _argmin_call(zqT, codebook)
    proj_cb = _cbproj_call(codebook, W_out.T, b_out.reshape(1, D))
    qq, qz = _gather_call()(idx, codebook, proj_cb)
    zero = jnp.float32(0.0)
    return (idx[:, None], qz, qq, qq[:, None, :], zero, zero, zero)


# speedup vs baseline: 1.6899x; 1.1568x over previous
import functools

import jax
import jax.numpy as jnp
from jax import lax
from jax.experimental import pallas as pl
from jax.experimental.pallas import tpu as pltpu
from jax.experimental.pallas import tpu_sc as plsc

B = 16384
D = 512
DQ = 256
K = 8192

TB = 512
WINDOWS = [(0, 2736), (2736, 5472), (5472, 8192)]


def _zq_body(ze_ref, winT_ref, bin_ref, zqT_ref):
    zq = jnp.dot(ze_ref[...], winT_ref[...],
                 preferred_element_type=jnp.float32) + bin_ref[...]
    zqT_ref[...] = zq.T


def _zq_call(ze, winT, bin2d):
    return pl.pallas_call(
        _zq_body,
        grid=(B // TB,),
        in_specs=[
            pl.BlockSpec((TB, D), lambda i: (i, 0)),
            pl.BlockSpec((D, DQ), lambda i: (0, 0)),
            pl.BlockSpec((1, DQ), lambda i: (0, 0)),
        ],
        out_specs=pl.BlockSpec((DQ, TB), lambda i: (0, i)),
        out_shape=jax.ShapeDtypeStruct((DQ, B), jnp.float32),
    )(ze, winT, bin2d)


def _argmin_body(zqT_ref, cb_ref, out_ref):
    zqT = zqT_ref[...]
    zsq = jnp.sum(zqT * zqT, axis=0, keepdims=True)
    zq2 = zqT + zqT
    acc_v = jnp.full((1, TB), jnp.inf, jnp.float32)
    acc_i = jnp.zeros((1, TB), jnp.int32)
    kio_full = lax.broadcasted_iota(jnp.int32, (WINDOWS[0][1], TB), 0)
    for lo, hi in WINDOWS:
        cbw = cb_ref[pl.ds(lo, hi - lo), :]
        csq = jnp.sum(cbw * cbw, axis=1, keepdims=True)
        s2 = lax.dot_general(cbw, zq2, (((1,), (0,)), ((), ())),
                             preferred_element_type=jnp.float32)
        dist = (zsq + csq) - s2
        kio = lax.slice(kio_full, (0, 0), (hi - lo, TB))
        mw = jnp.min(dist, axis=0, keepdims=True)
        iw = jnp.min(jnp.where(dist == mw, kio, K), axis=0, keepdims=True) + lo
        upd = mw < acc_v
        acc_i = jnp.where(upd, iw, acc_i)
        mwq = mw.astype(jnp.bfloat16).astype(jnp.float32)
        acc_v = jnp.where(upd, mwq, acc_v)
    out_ref[...] = acc_i.reshape(1, 1, TB)


def _argmin_call(zqT, codebook):
    out = pl.pallas_call(
        _argmin_body,
        grid=(B // TB,),
        in_specs=[
            pl.BlockSpec((DQ, TB), lambda i: (0, i)),
            pl.BlockSpec((K, DQ), lambda i: (0, 0)),
        ],
        out_specs=pl.BlockSpec((1, 1, TB), lambda i: (i, 0, 0)),
        out_shape=jax.ShapeDtypeStruct((B // TB, 1, TB), jnp.int32),
    )(zqT, codebook)
    return out.reshape(B)


CB = 512


def _cbproj_body(cb_ref, woutT_ref, bout_ref, p_ref):
    p_ref[...] = jnp.dot(cb_ref[...], woutT_ref[...],
                         preferred_element_type=jnp.float32) + bout_ref[...]


def _cbproj_call(codebook, woutT, bout2d):
    return pl.pallas_call(
        _cbproj_body,
        grid=(K // CB,),
        in_specs=[
            pl.BlockSpec((CB, DQ), lambda i: (i, 0)),
            pl.BlockSpec((DQ, D), lambda i: (0, 0)),
            pl.BlockSpec((1, D), lambda i: (0, 0)),
        ],
        out_specs=pl.BlockSpec((CB, D), lambda i: (i, 0)),
        out_shape=jax.ShapeDtypeStruct((K, D), jnp.float32),
    )(codebook, woutT, bout2d)


NC = 2
NS = 16
NW = NC * NS
BPW = B // NW
CH = 64
NCH = BPW // CH


def _gather_body(idx_hbm, cb_hbm, p_hbm, qq_hbm, qz_hbm,
                 idx_v, r1a, r1b, r2a, r2b, s1a, s1b, s2a, s2b):
    wid = lax.axis_index("s") * NC + lax.axis_index("c")
    base = wid * BPW
    rows1, rows2 = [r1a, r1b], [r2a, r2b]
    sem1, sem2 = [s1a, s1b], [s2a, s2b]
    pltpu.sync_copy(idx_hbm.at[pl.ds(base, BPW)], idx_v)
    cps = [None, None]
    for c in range(NCH + 1):
        if c < NCH:
            b = c % 2
            sl = idx_v.at[pl.ds(c * CH, CH)]
            cps[b] = (pltpu.async_copy(cb_hbm.at[sl], rows1[b], sem1[b]),
                      pltpu.async_copy(p_hbm.at[sl], rows2[b], sem2[b]))
        if c >= 1:
            p = (c - 1) % 2
            off = base + (c - 1) * CH
            cps[p][0].wait()
            pltpu.sync_copy(rows1[p], qq_hbm.at[pl.ds(off, CH)])
            cps[p][1].wait()
            pltpu.sync_copy(rows2[p], qz_hbm.at[pl.ds(off, CH)])


@functools.cache
def _gather_call():
    return pl.kernel(
        _gather_body,
        mesh=plsc.VectorSubcoreMesh(core_axis_name="c", subcore_axis_name="s"),
        out_type=[
            jax.ShapeDtypeStruct((B, DQ), jnp.float32),
            jax.ShapeDtypeStruct((B, D), jnp.float32),
        ],
        scratch_types=[
            pltpu.VMEM((BPW,), jnp.int32),
            pltpu.VMEM((CH, DQ), jnp.float32),
            pltpu.VMEM((CH, DQ), jnp.float32),
            pltpu.VMEM((CH, D), jnp.float32),
            pltpu.VMEM((CH, D), jnp.float32),
            pltpu.SemaphoreType.DMA,
            pltpu.SemaphoreType.DMA,
            pltpu.SemaphoreType.DMA,
            pltpu.SemaphoreType.DMA,
        ],
    )


def kernel(ze, W_in, b_in, codebook, W_out, b_out):
    zqT = _zq_call(ze, W_in.T, b_in.reshape(1, DQ))
    idx = _argmin_call(zqT, codebook)
    proj_cb = _cbproj_call(codebook, W_out.T, b_out.reshape(1, D))
    qq, qz = _gather_call()(idx, codebook, proj_cb)
    zero = jnp.float32(0.0)
    return (idx[:, None], qz, qq, qq[:, None, :], zero, zero, zero)


# speedup vs baseline: 1.8346x; 1.0857x over previous
import functools

import jax
import jax.numpy as jnp
from jax import lax
from jax.experimental import pallas as pl
from jax.experimental.pallas import tpu as pltpu
from jax.experimental.pallas import tpu_sc as plsc

B = 16384
D = 512
DQ = 256
K = 8192

TB = 1024
WINDOWS = [(0, 2736), (2736, 5472), (5472, 8192)]


def _zq_body(ze_ref, winT_ref, bin_ref, zqT_ref):
    zq = jnp.dot(ze_ref[...], winT_ref[...],
                 preferred_element_type=jnp.float32) + bin_ref[...]
    zqT_ref[...] = zq.T


def _zq_call(ze, winT, bin2d):
    return pl.pallas_call(
        _zq_body,
        grid=(B // TB,),
        in_specs=[
            pl.BlockSpec((TB, D), lambda i: (i, 0)),
            pl.BlockSpec((D, DQ), lambda i: (0, 0)),
            pl.BlockSpec((1, DQ), lambda i: (0, 0)),
        ],
        out_specs=pl.BlockSpec((DQ, TB), lambda i: (0, i)),
        out_shape=jax.ShapeDtypeStruct((DQ, B), jnp.float32),
    )(ze, winT, bin2d)


def _argmin_body(zqT_ref, cb_ref, out_ref):
    zqT = zqT_ref[...]
    zsq = jnp.sum(zqT * zqT, axis=0, keepdims=True)
    zq2 = zqT + zqT
    acc_v = jnp.full((1, TB), jnp.inf, jnp.float32)
    acc_i = jnp.zeros((1, TB), jnp.int32)
    kio_full = lax.broadcasted_iota(jnp.int32, (WINDOWS[0][1], TB), 0)
    for lo, hi in WINDOWS:
        cbw = cb_ref[pl.ds(lo, hi - lo), :]
        csq = jnp.sum(cbw * cbw, axis=1, keepdims=True)
        s2 = lax.dot_general(cbw, zq2, (((1,), (0,)), ((), ())),
                             preferred_element_type=jnp.float32)
        dist = (zsq + csq) - s2
        kio = lax.slice(kio_full, (0, 0), (hi - lo, TB))
        mw = jnp.min(dist, axis=0, keepdims=True)
        iw = jnp.min(jnp.where(dist == mw, kio, K), axis=0, keepdims=True) + lo
        upd = mw < acc_v
        acc_i = jnp.where(upd, iw, acc_i)
        mwq = mw.astype(jnp.bfloat16).astype(jnp.float32)
        acc_v = jnp.where(upd, mwq, acc_v)
    out_ref[...] = acc_i.reshape(1, 1, TB)


def _argmin_call(zqT, codebook):
    out = pl.pallas_call(
        _argmin_body,
        grid=(B // TB,),
        in_specs=[
            pl.BlockSpec((DQ, TB), lambda i: (0, i)),
            pl.BlockSpec((K, DQ), lambda i: (0, 0)),
        ],
        out_specs=pl.BlockSpec((1, 1, TB), lambda i: (i, 0, 0)),
        out_shape=jax.ShapeDtypeStruct((B // TB, 1, TB), jnp.int32),
    )(zqT, codebook)
    return out.reshape(B)


CB = 512


def _cbproj_body(cb_ref, woutT_ref, bout_ref, p_ref):
    p_ref[...] = jnp.dot(cb_ref[...], woutT_ref[...],
                         preferred_element_type=jnp.float32) + bout_ref[...]


def _cbproj_call(codebook, woutT, bout2d):
    return pl.pallas_call(
        _cbproj_body,
        grid=(K // CB,),
        in_specs=[
            pl.BlockSpec((CB, DQ), lambda i: (i, 0)),
            pl.BlockSpec((DQ, D), lambda i: (0, 0)),
            pl.BlockSpec((1, D), lambda i: (0, 0)),
        ],
        out_specs=pl.BlockSpec((CB, D), lambda i: (i, 0)),
        out_shape=jax.ShapeDtypeStruct((K, D), jnp.float32),
    )(codebook, woutT, bout2d)


NC = 2
NS = 16
NW = NC * NS
BPW = B // NW
CH = 64
NCH = BPW // CH


def _gather_body(idx_hbm, cb_hbm, p_hbm, qq_hbm, qz_hbm,
                 idx_v, r1a, r1b, r2a, r2b, s1a, s1b, s2a, s2b):
    wid = lax.axis_index("s") * NC + lax.axis_index("c")
    base = wid * BPW
    rows1, rows2 = [r1a, r1b], [r2a, r2b]
    sem1, sem2 = [s1a, s1b], [s2a, s2b]
    pltpu.sync_copy(idx_hbm.at[pl.ds(base, BPW)], idx_v)
    cps = [None, None]
    for c in range(NCH + 1):
        if c < NCH:
            b = c % 2
            sl = idx_v.at[pl.ds(c * CH, CH)]
            cps[b] = (pltpu.async_copy(cb_hbm.at[sl], rows1[b], sem1[b]),
                      pltpu.async_copy(p_hbm.at[sl], rows2[b], sem2[b]))
        if c >= 1:
            p = (c - 1) % 2
            off = base + (c - 1) * CH
            cps[p][0].wait()
            pltpu.sync_copy(rows1[p], qq_hbm.at[pl.ds(off, CH)])
            cps[p][1].wait()
            pltpu.sync_copy(rows2[p], qz_hbm.at[pl.ds(off, CH)])


@functools.cache
def _gather_call():
    return pl.kernel(
        _gather_body,
        mesh=plsc.VectorSubcoreMesh(core_axis_name="c", subcore_axis_name="s"),
        out_type=[
            jax.ShapeDtypeStruct((B, DQ), jnp.float32),
            jax.ShapeDtypeStruct((B, D), jnp.float32),
        ],
        scratch_types=[
            pltpu.VMEM((BPW,), jnp.int32),
            pltpu.VMEM((CH, DQ), jnp.float32),
            pltpu.VMEM((CH, DQ), jnp.float32),
            pltpu.VMEM((CH, D), jnp.float32),
            pltpu.VMEM((CH, D), jnp.float32),
            pltpu.SemaphoreType.DMA,
            pltpu.SemaphoreType.DMA,
            pltpu.SemaphoreType.DMA,
            pltpu.SemaphoreType.DMA,
        ],
    )


def kernel(ze, W_in, b_in, codebook, W_out, b_out):
    zqT = _zq_call(ze, W_in.T, b_in.reshape(1, DQ))
    idx = _argmin_call(zqT, codebook)
    proj_cb = _cbproj_call(codebook, W_out.T, b_out.reshape(1, D))
    qq, qz = _gather_call()(idx, codebook, proj_cb)
    zero = jnp.float32(0.0)
    return (idx[:, None], qz, qq, qq[:, None, :], zero, zero, zero)
